# Initial kernel scaffold; baseline (speedup 1.0000x reference)
import jax
import jax.numpy as jnp
from jax.experimental import pallas as pl


def kernel(x, edge_index, batch, W1, b1, W2, b2, Wlin, blin):
    raise NotImplementedError("write your pallas kernel here")


# speedup vs baseline: 19.3612x; 19.3612x over previous
import functools

import jax
import jax.numpy as jnp
from jax import lax
from jax.experimental import pallas as pl
from jax.experimental.pallas import tpu as pltpu
from jax.experimental.pallas import tpu_sc as plsc

NC = 2
NS = 16
KCH = 80
RPT = 640


def _sc_mesh():
    return plsc.VectorSubcoreMesh(core_axis_name="c", subcore_axis_name="s")


def _own_rows(n, s):
    last = n - (NS - 1) * RPT
    return s * RPT, RPT if s < NS - 1 else last


def _degree_partials(dst3, ones_k, zeros_r, n):
    per_w = dst3.shape[1]

    @functools.partial(
        pl.kernel,
        out_type=jax.ShapeDtypeStruct((NC, n, 1), jnp.float32),
        mesh=_sc_mesh(),
        scratch_types=[
            pltpu.VMEM((per_w, KCH), jnp.int32),
            pltpu.VMEM((KCH, 1), jnp.float32),
            pltpu.VMEM_SHARED((n, 1), jnp.float32),
        ],
    )
    def k(dst_hbm, ones_hbm, zeros_hbm, out_hbm, didx, onesb, acc):
        c = lax.axis_index("c")
        s = lax.axis_index("s")
        wid = c * NS + s
        pltpu.sync_copy(ones_hbm, onesb)
        pltpu.sync_copy(dst_hbm.at[wid], didx)

        @pl.when(s < NS - 1)
        def _():
            pltpu.sync_copy(zeros_hbm, acc.at[pl.ds(s * RPT, RPT), :])

        @pl.when(s == NS - 1)
        def _():
            base, size = _own_rows(n, NS - 1)
            pltpu.sync_copy(zeros_hbm.at[pl.ds(0, size), :],
                            acc.at[pl.ds(base, size), :])

        plsc.subcore_barrier()

        def sub(j, _):
            pltpu.sync_copy(onesb, acc.at[didx.at[j]], add=True)
            return 0

        lax.fori_loop(0, per_w, sub, 0)
        plsc.subcore_barrier()

        @pl.when(s < NS - 1)
        def _():
            pltpu.sync_copy(acc.at[pl.ds(s * RPT, RPT), :],
                            out_hbm.at[c, pl.ds(s * RPT, RPT), :])

        @pl.when(s == NS - 1)
        def _():
            base, size = _own_rows(n, NS - 1)
            pltpu.sync_copy(acc.at[pl.ds(base, size), :],
                            out_hbm.at[c, pl.ds(base, size), :])

    return k(dst3, ones_k, zeros_r)


def _aggregate_partials(g, src3, dst3, zeros_r, n, h):
    per_w = src3.shape[1]

    @functools.partial(
        pl.kernel,
        out_type=jax.ShapeDtypeStruct((NC, n, h), jnp.float32),
        mesh=_sc_mesh(),
        scratch_types=[
            pltpu.VMEM((per_w, KCH), jnp.int32),
            pltpu.VMEM((per_w, KCH), jnp.int32),
            pltpu.VMEM((KCH, h), jnp.float32),
            pltpu.VMEM_SHARED((n, h), jnp.float32),
            pltpu.SemaphoreType.DMA,
        ],
    )
    def k(g_hbm, src_hbm, dst_hbm, zeros_hbm, out_hbm, sidx, didx, rows, acc,
          sem):
        c = lax.axis_index("c")
        s = lax.axis_index("s")
        wid = c * NS + s
        pltpu.sync_copy(src_hbm.at[wid], sidx)
        pltpu.sync_copy(dst_hbm.at[wid], didx)

        @pl.when(s < NS - 1)
        def _():
            pltpu.sync_copy(zeros_hbm, acc.at[pl.ds(s * RPT, RPT), :])

        @pl.when(s == NS - 1)
        def _():
            base, size = _own_rows(n, NS - 1)
            pltpu.sync_copy(zeros_hbm.at[pl.ds(0, size), :],
                            acc.at[pl.ds(base, size), :])

        plsc.subcore_barrier()

        def sub(j, _):
            pltpu.async_copy(g_hbm.at[sidx.at[j]], rows, sem).wait()
            pltpu.sync_copy(rows, acc.at[didx.at[j]], add=True)
            return 0

        lax.fori_loop(0, per_w, sub, 0)
        plsc.subcore_barrier()

        @pl.when(s < NS - 1)
        def _():
            pltpu.sync_copy(acc.at[pl.ds(s * RPT, RPT), :],
                            out_hbm.at[c, pl.ds(s * RPT, RPT), :])

        @pl.when(s == NS - 1)
        def _():
            base, size = _own_rows(n, NS - 1)
            pltpu.sync_copy(acc.at[pl.ds(base, size), :],
                            out_hbm.at[c, pl.ds(base, size), :])

    return k(g, src3, dst3, zeros_r)


def _tc_pre(p, x, w1, rb):
    n, d = x.shape
    hh = w1.shape[1]
    grid = n // rb

    def body(p_ref, x_ref, w_ref, out_ref):
        pp = p_ref[...]
        dv = lax.rsqrt(pp[0] + pp[1] + 1.0)
        out_ref[...] = dv * jnp.dot(x_ref[...], w_ref[...],
                                    preferred_element_type=jnp.float32)

    return pl.pallas_call(
        body,
        grid=(grid,),
        in_specs=[
            pl.BlockSpec((2, rb, 1), lambda i: (0, i, 0)),
            pl.BlockSpec((rb, d), lambda i: (i, 0)),
            pl.BlockSpec((d, hh), lambda i: (0, 0)),
        ],
        out_specs=pl.BlockSpec((rb, hh), lambda i: (i, 0)),
        out_shape=jax.ShapeDtypeStruct((n, hh), jnp.float32),
    )(p, x, w1)


def _tc_mid(p, s1, g1, b1, w2, rb):
    n, hh = g1.shape

    def body(p_ref, s_ref, g_ref, b_ref, w_ref, out_ref):
        pp = p_ref[...]
        dv = lax.rsqrt(pp[0] + pp[1] + 1.0)
        ss = s_ref[...]
        h1 = jax.nn.relu(dv * (ss[0] + ss[1] + g_ref[...]) + b_ref[...])
        out_ref[...] = dv * jnp.dot(h1, w_ref[...],
                                    preferred_element_type=jnp.float32)

    return pl.pallas_call(
        body,
        grid=(n // rb,),
        in_specs=[
            pl.BlockSpec((2, rb, 1), lambda i: (0, i, 0)),
            pl.BlockSpec((2, rb, hh), lambda i: (0, i, 0)),
            pl.BlockSpec((rb, hh), lambda i: (i, 0)),
            pl.BlockSpec((1, hh), lambda i: (0, 0)),
            pl.BlockSpec((hh, hh), lambda i: (0, 0)),
        ],
        out_specs=pl.BlockSpec((rb, hh), lambda i: (i, 0)),
        out_shape=jax.ShapeDtypeStruct((n, hh), jnp.float32),
    )(p, s1, g1, b1, w2)


def _tc_final(p, s2, g2, b2, batch3, wlin, blin, rb, nseg):
    n, hh = g2.shape
    cc = wlin.shape[1]
    grid = n // rb

    def body(p_ref, s_ref, g_ref, b_ref, bat_ref, wl_ref, bl_ref, out_ref,
             acc, cnt):
        i = pl.program_id(0)

        @pl.when(i == 0)
        def _():
            acc[...] = jnp.zeros_like(acc)
            cnt[...] = jnp.zeros_like(cnt)

        pp = p_ref[...]
        dv = lax.rsqrt(pp[0] + pp[1] + 1.0)
        ss = s_ref[...]
        h2 = jax.nn.relu(dv * (ss[0] + ss[1] + g_ref[...]) + b_ref[...])
        bat = bat_ref[...].reshape(1, rb)
        seg = lax.broadcasted_iota(jnp.int32, (nseg, rb), 0)
        mask = (seg == jnp.broadcast_to(bat, (nseg, rb))).astype(jnp.float32)
        acc[...] += jnp.dot(mask, h2, preferred_element_type=jnp.float32)
        cnt[...] += jnp.sum(mask, axis=1, keepdims=True)

        @pl.when(i == grid - 1)
        def _():
            pooled = acc[...] / jnp.maximum(cnt[...], 1.0)
            out_ref[...] = jnp.dot(pooled, wl_ref[...],
                                   preferred_element_type=jnp.float32) \
                + bl_ref[...]

    return pl.pallas_call(
        body,
        grid=(grid,),
        in_specs=[
            pl.BlockSpec((2, rb, 1), lambda i: (0, i, 0)),
            pl.BlockSpec((2, rb, hh), lambda i: (0, i, 0)),
            pl.BlockSpec((rb, hh), lambda i: (i, 0)),
            pl.BlockSpec((1, hh), lambda i: (0, 0)),
            pl.BlockSpec((1, 1, rb), lambda i: (i, 0, 0)),
            pl.BlockSpec((hh, cc), lambda i: (0, 0)),
            pl.BlockSpec((1, cc), lambda i: (0, 0)),
        ],
        out_specs=pl.BlockSpec((nseg, cc), lambda i: (0, 0)),
        out_shape=jax.ShapeDtypeStruct((nseg, cc), jnp.float32),
        scratch_shapes=[
            pltpu.VMEM((nseg, hh), jnp.float32),
            pltpu.VMEM((nseg, 1), jnp.float32),
        ],
    )(p, s2, g2, b2, batch3, wlin, blin)


def kernel(x, edge_index, batch, W1, b1, W2, b2, Wlin, blin):
    n, d = x.shape
    e = edge_index.shape[1]
    hh = W1.shape[1]
    nseg = 64
    rb = 1000

    per_w = e // (NC * NS * KCH)
    src2 = edge_index[0].reshape(NC * NS, per_w, KCH)
    dst2 = edge_index[1].reshape(NC * NS, per_w, KCH)
    ones_k = jnp.ones((KCH, 1), jnp.float32)
    zeros_d = jnp.zeros((RPT, 1), jnp.float32)
    zeros_r = jnp.zeros((RPT, hh), jnp.float32)
    batch3 = batch.reshape(n // rb, 1, rb)
    b1r = b1.reshape(1, hh)
    b2r = b2.reshape(1, hh)
    blr = blin.reshape(1, -1)

    p = _degree_partials(dst2, ones_k, zeros_d, n)
    g1 = _tc_pre(p, x, W1, rb)
    s1 = _aggregate_partials(g1, src2, dst2, zeros_r, n, hh)
    g2 = _tc_mid(p, s1, g1, b1r, W2, rb)
    s2 = _aggregate_partials(g2, src2, dst2, zeros_r, n, hh)
    return _tc_final(p, s2, g2, b2r, batch3, Wlin, blr, rb, nseg)


# speedup vs baseline: 21.2976x; 1.1000x over previous
import functools

import jax
import jax.numpy as jnp
from jax import lax
from jax.experimental import pallas as pl
from jax.experimental.pallas import tpu as pltpu
from jax.experimental.pallas import tpu_sc as plsc

NC = 2
NS = 16
KCH = 80
RPT = 640


def _sc_mesh():
    return plsc.VectorSubcoreMesh(core_axis_name="c", subcore_axis_name="s")


def _own_rows(n, s):
    last = n - (NS - 1) * RPT
    return s * RPT, RPT if s < NS - 1 else last


def _degree_partials(dst3, ones_k, zeros_r, n):
    per_w = dst3.shape[1]

    @functools.partial(
        pl.kernel,
        out_type=jax.ShapeDtypeStruct((NC, n, 1), jnp.float32),
        mesh=_sc_mesh(),
        scratch_types=[
            pltpu.VMEM((per_w, KCH), jnp.int32),
            pltpu.VMEM((KCH, 1), jnp.float32),
            pltpu.VMEM_SHARED((n + 8, 1), jnp.float32),
        ],
    )
    def k(dst_hbm, ones_hbm, zeros_hbm, out_hbm, didx, onesb, acc):
        c = lax.axis_index("c")
        s = lax.axis_index("s")
        wid = c * NS + s
        pltpu.sync_copy(ones_hbm, onesb)
        pltpu.sync_copy(dst_hbm.at[wid], didx)

        @pl.when(s < NS - 1)
        def _():
            pltpu.sync_copy(zeros_hbm, acc.at[pl.ds(s * RPT, RPT), :])

        @pl.when(s == NS - 1)
        def _():
            base, size = _own_rows(n, NS - 1)
            pltpu.sync_copy(zeros_hbm.at[pl.ds(0, size), :],
                            acc.at[pl.ds(base, size), :])

        plsc.subcore_barrier()

        def sub(j, _):
            pltpu.sync_copy(onesb, acc.at[didx.at[j]], add=True)
            return 0

        lax.fori_loop(0, per_w, sub, 0)
        plsc.subcore_barrier()

        @pl.when(s < NS - 1)
        def _():
            pltpu.sync_copy(acc.at[pl.ds(s * RPT, RPT), :],
                            out_hbm.at[c, pl.ds(s * RPT, RPT), :])

        @pl.when(s == NS - 1)
        def _():
            base, size = _own_rows(n, NS - 1)
            pltpu.sync_copy(acc.at[pl.ds(base, size), :],
                            out_hbm.at[c, pl.ds(base, size), :])

    return k(dst3, ones_k, zeros_r)


def _aggregate_partials(g, src3, dst3, zeros_r, n, h):
    ngrp = src3.shape[1]

    @functools.partial(
        pl.kernel,
        out_type=jax.ShapeDtypeStruct((NC, n, h), jnp.float32),
        mesh=_sc_mesh(),
        scratch_types=[
            pltpu.VMEM((8, KCH), jnp.int32),
            pltpu.VMEM((8, KCH), jnp.int32),
            pltpu.VMEM((KCH, h), jnp.float32),
            pltpu.VMEM((KCH, h), jnp.float32),
            pltpu.VMEM_SHARED((n + 8, h), jnp.float32),
            pltpu.SemaphoreType.DMA,
            pltpu.SemaphoreType.DMA,
        ],
    )
    def k(g_hbm, src_hbm, dst_hbm, zeros_hbm, out_hbm, sidx, didx, rows_a,
          rows_b, acc, sem_a, sem_b):
        c = lax.axis_index("c")
        s = lax.axis_index("s")
        wid = c * NS + s

        b0, sz = _own_rows(n, NS - 1)

        @pl.when(s < NS - 1)
        def _():
            pltpu.sync_copy(zeros_hbm, acc.at[pl.ds(s * RPT, RPT), :])

        @pl.when(s == NS - 1)
        def _():
            pltpu.sync_copy(zeros_hbm.at[pl.ds(0, sz), :],
                            acc.at[pl.ds(b0, sz), :])
            pltpu.sync_copy(zeros_hbm.at[pl.ds(0, 8), :],
                            acc.at[pl.ds(n, 8), :])

        plsc.subcore_barrier()

        def grp(gi, _):
            pltpu.sync_copy(src_hbm.at[wid, gi], sidx)
            pltpu.sync_copy(dst_hbm.at[wid, gi], didx)
            bufs = (rows_a, rows_b)
            sems = (sem_a, sem_b)
            pend = pltpu.async_copy(g_hbm.at[sidx.at[0]], rows_a, sem_a)
            for jj in range(8):
                rb_ = bufs[jj % 2]
                pend.wait()
                if jj + 1 < 8:
                    pend = pltpu.async_copy(g_hbm.at[sidx.at[jj + 1]],
                                            bufs[(jj + 1) % 2],
                                            sems[(jj + 1) % 2])
                pltpu.sync_copy(rb_, acc.at[didx.at[jj]], add=True)
            return 0

        lax.fori_loop(0, ngrp, grp, 0)
        plsc.subcore_barrier()

        @pl.when(s < NS - 1)
        def _():
            pltpu.sync_copy(acc.at[pl.ds(s * RPT, RPT), :],
                            out_hbm.at[c, pl.ds(s * RPT, RPT), :])

        @pl.when(s == NS - 1)
        def _():
            base, size = _own_rows(n, NS - 1)
            pltpu.sync_copy(acc.at[pl.ds(base, size), :],
                            out_hbm.at[c, pl.ds(base, size), :])

    return k(g, src3, dst3, zeros_r)


def _tc_pre(p, x, w1, rb):
    n, d = x.shape
    hh = w1.shape[1]
    grid = n // rb

    def body(p_ref, x_ref, w_ref, out_ref):
        pp = p_ref[...]
        dv = lax.rsqrt(pp[0] + pp[1] + 1.0)
        out_ref[...] = dv * jnp.dot(x_ref[...], w_ref[...],
                                    preferred_element_type=jnp.float32)

    return pl.pallas_call(
        body,
        grid=(grid,),
        in_specs=[
            pl.BlockSpec((2, rb, 1), lambda i: (0, i, 0)),
            pl.BlockSpec((rb, d), lambda i: (i, 0)),
            pl.BlockSpec((d, hh), lambda i: (0, 0)),
        ],
        out_specs=pl.BlockSpec((rb, hh), lambda i: (i, 0)),
        out_shape=jax.ShapeDtypeStruct((n, hh), jnp.float32),
    )(p, x, w1)


def _tc_mid(p, s1, g1, b1, w2, rb):
    n, hh = g1.shape

    def body(p_ref, s_ref, g_ref, b_ref, w_ref, out_ref):
        pp = p_ref[...]
        dv = lax.rsqrt(pp[0] + pp[1] + 1.0)
        ss = s_ref[...]
        h1 = jax.nn.relu(dv * (ss[0] + ss[1] + g_ref[...]) + b_ref[...])
        out_ref[...] = dv * jnp.dot(h1, w_ref[...],
                                    preferred_element_type=jnp.float32)

    return pl.pallas_call(
        body,
        grid=(n // rb,),
        in_specs=[
            pl.BlockSpec((2, rb, 1), lambda i: (0, i, 0)),
            pl.BlockSpec((2, rb, hh), lambda i: (0, i, 0)),
            pl.BlockSpec((rb, hh), lambda i: (i, 0)),
            pl.BlockSpec((1, hh), lambda i: (0, 0)),
            pl.BlockSpec((hh, hh), lambda i: (0, 0)),
        ],
        out_specs=pl.BlockSpec((rb, hh), lambda i: (i, 0)),
        out_shape=jax.ShapeDtypeStruct((n, hh), jnp.float32),
    )(p, s1, g1, b1, w2)


def _tc_final(p, s2, g2, b2, batch3, wlin, blin, rb, nseg):
    n, hh = g2.shape
    cc = wlin.shape[1]
    grid = n // rb

    def body(p_ref, s_ref, g_ref, b_ref, bat_ref, wl_ref, bl_ref, out_ref,
             acc, cnt):
        i = pl.program_id(0)

        @pl.when(i == 0)
        def _():
            acc[...] = jnp.zeros_like(acc)
            cnt[...] = jnp.zeros_like(cnt)

        pp = p_ref[...]
        dv = lax.rsqrt(pp[0] + pp[1] + 1.0)
        ss = s_ref[...]
        h2 = jax.nn.relu(dv * (ss[0] + ss[1] + g_ref[...]) + b_ref[...])
        bat = bat_ref[...].reshape(1, rb)
        seg = lax.broadcasted_iota(jnp.int32, (nseg, rb), 0)
        mask = (seg == jnp.broadcast_to(bat, (nseg, rb))).astype(jnp.float32)
        acc[...] += jnp.dot(mask, h2, preferred_element_type=jnp.float32)
        cnt[...] += jnp.sum(mask, axis=1, keepdims=True)

        @pl.when(i == grid - 1)
        def _():
            pooled = acc[...] / jnp.maximum(cnt[...], 1.0)
            out_ref[...] = jnp.dot(pooled, wl_ref[...],
                                   preferred_element_type=jnp.float32) \
                + bl_ref[...]

    return pl.pallas_call(
        body,
        grid=(grid,),
        in_specs=[
            pl.BlockSpec((2, rb, 1), lambda i: (0, i, 0)),
            pl.BlockSpec((2, rb, hh), lambda i: (0, i, 0)),
            pl.BlockSpec((rb, hh), lambda i: (i, 0)),
            pl.BlockSpec((1, hh), lambda i: (0, 0)),
            pl.BlockSpec((1, 1, rb), lambda i: (i, 0, 0)),
            pl.BlockSpec((hh, cc), lambda i: (0, 0)),
            pl.BlockSpec((1, cc), lambda i: (0, 0)),
        ],
        out_specs=pl.BlockSpec((nseg, cc), lambda i: (0, 0)),
        out_shape=jax.ShapeDtypeStruct((nseg, cc), jnp.float32),
        scratch_shapes=[
            pltpu.VMEM((nseg, hh), jnp.float32),
            pltpu.VMEM((nseg, 1), jnp.float32),
        ],
    )(p, s2, g2, b2, batch3, wlin, blin)


def kernel(x, edge_index, batch, W1, b1, W2, b2, Wlin, blin):
    n, d = x.shape
    e = edge_index.shape[1]
    hh = W1.shape[1]
    nseg = 64
    rb = 1000

    per_w = e // (NC * NS * KCH)
    per_wp = (per_w + 7) // 8 * 8
    npad = per_wp - per_w
    src2 = edge_index[0].reshape(NC * NS, per_w, KCH)
    dst2 = edge_index[1].reshape(NC * NS, per_w, KCH)
    if npad:
        spread = (jnp.arange(NC * NS * npad * KCH, dtype=jnp.int32)
                  .reshape(NC * NS, npad, KCH))
        src2 = jnp.concatenate([src2, spread * 2003 % n], axis=1)
        dst2 = jnp.concatenate([dst2, n + spread % 8], axis=1)
    src4 = src2.reshape(NC * NS, per_wp // 8, 8, KCH)
    dst4 = dst2.reshape(NC * NS, per_wp // 8, 8, KCH)
    zeros_r = jnp.zeros((RPT, hh), jnp.float32)
    ones_k = jnp.ones((KCH, 1), jnp.float32)
    zeros_d = jnp.zeros((RPT, 1), jnp.float32)
    batch3 = batch.reshape(n // rb, 1, rb)
    b1r = b1.reshape(1, hh)
    b2r = b2.reshape(1, hh)
    blr = blin.reshape(1, -1)

    p = _degree_partials(dst2, ones_k, zeros_d, n)
    g1 = _tc_pre(p, x, W1, rb)
    s1 = _aggregate_partials(g1, src4, dst4, zeros_r, n, hh)
    g2 = _tc_mid(p, s1, g1, b1r, W2, rb)
    s2 = _aggregate_partials(g2, src4, dst4, zeros_r, n, hh)
    return _tc_final(p, s2, g2, b2r, batch3, Wlin, blr, rb, nseg)
